# Initial kernel scaffold; baseline (speedup 1.0000x reference)
import jax
import jax.numpy as jnp
from jax.experimental import pallas as pl


def kernel(preds, targets, masks, img_w, img_h):
    raise NotImplementedError("write your pallas kernel here")


# speedup vs baseline: 113.3116x; 113.3116x over previous
import jax
import jax.numpy as jnp
from jax.experimental import pallas as pl
from jax.experimental.pallas import tpu as pltpu

_W_REG = 3.0
_LEN = 15.0
_K = 4


def _assign_kernel(scal_ref, predst_ref, tgt_ref, tgtt_ref, masks_ref, out_ref):
    wm1 = scal_ref[0]
    hm1 = scal_ref[1]
    wf = scal_ref[2]
    T = tgt_ref.shape[1]
    P = predst_ref.shape[2]
    O = predst_ref.shape[1] - 6

    ppts = predst_ref[0, 6:, :] * wm1
    sx_p = predst_ref[0, 2:3, :] * hm1
    sy_p = predst_ref[0, 3:4, :] * wm1
    th_p = predst_ref[0, 4:5, :]
    logit0 = predst_ref[0, 0:1, :]
    logit1 = predst_ref[0, 1:2, :]

    tsx = tgt_ref[0, :, 2:3] * hm1
    tsy = tgt_ref[0, :, 3:4] * wm1
    tth = tgt_ref[0, :, 4:5]
    gt0 = tgt_ref[0, :, 1:2].astype(jnp.int32) == 0
    tpts_bt = tgt_ref[0, :, 6:] * wm1
    valid_bt = (tpts_bt >= 0.0) & (tpts_bt < wf)
    nvalid = jnp.sum(valid_bt.astype(jnp.float32), axis=1, keepdims=True)

    tpts_T = tgtt_ref[0, 6:, :] * wm1
    valid_T = ((tpts_T >= 0.0) & (tpts_T < wf)).astype(jnp.float32)

    rows = []
    for t in range(T):
        tcol = tpts_T[:, t:t + 1]
        mcol = valid_T[:, t:t + 1]
        d = jnp.abs(ppts - tcol) * mcol
        rows.append(jnp.sum(d, axis=0, keepdims=True))
    S = jnp.concatenate(rows, axis=0)

    lens = jnp.maximum(nvalid, 1.0)
    dist = S / lens
    dmax = jnp.maximum(jnp.max(dist), 1e-8)
    ds = 1.0 - dist / dmax + 0.01

    sd = jnp.sqrt(jnp.maximum((sx_p - tsx) ** 2 + (sy_p - tsy) ** 2, 0.0))
    smax = jnp.maximum(jnp.max(sd), 1e-8)
    ss = 1.0 - sd / smax + 0.01

    td = jnp.abs(th_p - tth) * 180.0
    tmax = jnp.maximum(jnp.max(td), 1e-8)
    ts = 1.0 - td / tmax + 0.01

    reg = (jnp.maximum(ds, 0.001) * jnp.maximum(ss, 0.001)
           * jnp.maximum(ts, 0.001))

    def focal(lg):
        p = jax.nn.sigmoid(lg)
        neg = -jnp.log(1.0 - p + 1e-12) * 0.75 * (p * p)
        pos = -jnp.log(p + 1e-12) * 0.25 * ((1.0 - p) * (1.0 - p))
        return pos - neg
    g0 = focal(logit0)
    g1 = focal(logit1)
    cls_cost = jnp.where(gt0, g0, g1)

    cost = -(reg * reg) * _W_REG + cls_cost
    vmask = masks_ref[0] > 0.0
    cost = jnp.where(vmask, cost, 1e8)

    two_l = 2.0 * _LEN
    liou = (two_l * nvalid - S) / (two_l * nvalid + S + 1e-9)
    liou = jnp.where(vmask, liou, 0.0)

    iota_p = jax.lax.broadcasted_iota(jnp.int32, (T, P), 1).astype(jnp.float32)
    iota_t = jax.lax.broadcasted_iota(jnp.int32, (T, P), 0).astype(jnp.float32)
    big_p = float(P)

    a = liou
    sum_top = jnp.zeros((T, 1), jnp.float32)
    for _ in range(_K):
        mx = jnp.max(a, axis=1, keepdims=True)
        sum_top = sum_top + mx
        fid = jnp.min(jnp.where(a >= mx, iota_p, big_p), axis=1, keepdims=True)
        a = jnp.where(iota_p == fid, -3e38, a)
    ks = jnp.maximum(sum_top.astype(jnp.int32), 1)
    ks = jnp.where(vmask, ks, 0)

    c = cost
    idxs = []
    for _ in range(_K):
        mn = jnp.min(c, axis=1, keepdims=True)
        fid = jnp.min(jnp.where(c <= mn, iota_p, big_p), axis=1, keepdims=True)
        idxs.append(fid)
        c = jnp.where(iota_p == fid, 3e38, c)

    matching = jnp.zeros((T, P), jnp.float32)
    for r in range(_K):
        sel = (ks > r).astype(jnp.float32)
        matching = matching + jnp.where(iota_p == idxs[r], 1.0, 0.0) * sel

    counts = jnp.sum(matching, axis=0, keepdims=True)
    conflict = counts > 1.0
    has_match = counts > 0.5

    inf = jnp.float32(jnp.inf)
    mc = jnp.where(matching > 0.5, cost, inf)
    mnc = jnp.min(mc, axis=0, keepdims=True)
    big_t = float(T)
    best_t = jnp.min(jnp.where(mc <= mnc, iota_t, big_t), axis=0, keepdims=True)
    first_t = jnp.min(jnp.where(matching > 0.5, iota_t, big_t), axis=0,
                      keepdims=True)

    matched = jnp.where(conflict, best_t, first_t)
    matched = jnp.where(has_match, matched, -1.0)
    out_ref[0] = matched.astype(jnp.int32)


def kernel(preds, targets, masks, img_w, img_h):
    B, P, D = preds.shape
    T = targets.shape[1]
    wf = jnp.asarray(img_w, jnp.float32)
    hf = jnp.asarray(img_h, jnp.float32)
    scal = jnp.stack([wf - 1.0, hf - 1.0, wf])

    predst = jnp.transpose(preds, (0, 2, 1))
    tgtt = jnp.transpose(targets, (0, 2, 1))
    masks3 = masks.reshape(B, T, 1)

    matched3 = pl.pallas_call(
        _assign_kernel,
        grid=(B,),
        in_specs=[
            pl.BlockSpec(memory_space=pltpu.SMEM),
            pl.BlockSpec((1, D, P), lambda b: (b, 0, 0)),
            pl.BlockSpec((1, T, D), lambda b: (b, 0, 0)),
            pl.BlockSpec((1, D, T), lambda b: (b, 0, 0)),
            pl.BlockSpec((1, T, 1), lambda b: (b, 0, 0)),
        ],
        out_specs=pl.BlockSpec((1, 1, P), lambda b: (b, 0, 0)),
        out_shape=jax.ShapeDtypeStruct((B, 1, P), jnp.int32),
    )(scal, predst, targets, tgtt, masks3)

    matched = matched3.reshape(B, P)
    return (matched >= 0, matched)
